# Initial kernel scaffold; baseline (speedup 1.0000x reference)
import jax
import jax.numpy as jnp
from jax.experimental import pallas as pl


def kernel(out_mu, out_sigma, out_pi, loc1, scale1, y, pre_obs):
    raise NotImplementedError("write your pallas kernel here")


# speedup vs baseline: 3.7567x; 3.7567x over previous
import functools

import jax
import jax.numpy as jnp
from jax.experimental import pallas as pl
from jax.experimental.pallas import tpu as pltpu

EPS = 1e-06
PRED_LENGTH = 12
K = 20
T = 12
L = 2 * T


def _body(mu_ref, sg_ref, l1_ref, sc_ref, y_ref, pi_ref,
          ade1_ref, fde1_ref, ade2_ref, fde2_ref, sums_ref):
    j = pl.program_id(0)
    bb = y_ref.shape[0]
    y = y_ref[...]
    lane = jax.lax.broadcasted_iota(jnp.int32, (1, bb, L), 2)
    even = (lane % 2 == 0).astype(jnp.float32)
    last_pair = (lane == L - 2).astype(jnp.float32)
    kio = jax.lax.broadcasted_iota(jnp.int32, (K, bb), 0)

    def head(mu, sg):
        diff = mu - y[None]
        e = diff * diff
        s = e + pltpu.roll(e, shift=L - 1, axis=2)
        d = jnp.sqrt(s) * even
        l2 = jnp.sum(d, axis=2)
        f = jnp.sqrt(jnp.sum(s * last_pair, axis=2))
        mn = jnp.min(l2, axis=0, keepdims=True)
        am = jnp.min(jnp.where(l2 == mn, kio, K), axis=0)
        mnf = jnp.min(f, axis=0, keepdims=True)
        amf = jnp.min(jnp.where(f == mnf, kio, K), axis=0)
        amc = am[:, None]
        amfc = amf[:, None]
        best = jnp.zeros((bb, L), jnp.float32)
        bestf = jnp.zeros((bb, L), jnp.float32)
        bsg = jnp.zeros((bb, L), jnp.float32)
        for k in range(K):
            best = best + jnp.where(amc == k, mu[k], 0.0)
            bestf = bestf + jnp.where(amfc == k, mu[k], 0.0)
            bsg = bsg + jnp.where(amc == k, sg[k], 0.0)
        sc = jnp.maximum(bsg, EPS)
        nll = jnp.log(2.0 * sc) + jnp.abs(y - best) / sc
        return l2, best, bestf, jnp.sum(nll)

    l2_1, ade1, fde1, reg1 = head(mu_ref[...], sg_ref[...])
    _, ade2, fde2, reg2 = head(l1_ref[...], sc_ref[...])

    z = -l2_1 / PRED_LENGTH
    z = z - jnp.max(z, axis=0, keepdims=True)
    ez = jnp.exp(z)
    st = ez / jnp.sum(ez, axis=0, keepdims=True)
    pi = pi_ref[...]
    pim = pi - jnp.max(pi, axis=1, keepdims=True)
    lsm = pim - jnp.log(jnp.sum(jnp.exp(pim), axis=1, keepdims=True))
    m = jax.lax.dot(st, lsm, preferred_element_type=jnp.float32)
    keye = (jax.lax.broadcasted_iota(jnp.int32, (K, K), 0)
            == jax.lax.broadcasted_iota(jnp.int32, (K, K), 1)).astype(jnp.float32)
    tr = jnp.sum(m * keye)

    ade1_ref[...] = ade1
    fde1_ref[...] = fde1
    ade2_ref[...] = ade2
    fde2_ref[...] = fde2

    lane1 = jax.lax.broadcasted_iota(jnp.int32, (1, 128), 1)
    upd = (jnp.where(lane1 == 0, reg1, 0.0)
           + jnp.where(lane1 == 1, tr, 0.0)
           + jnp.where(lane1 == 2, reg2, 0.0))

    @pl.when(j == 0)
    def _():
        sums_ref[...] = jnp.zeros_like(sums_ref)

    sums_ref[...] += upd


@functools.partial(jax.jit, static_argnames=())
def kernel(out_mu, out_sigma, out_pi, loc1, scale1, y, pre_obs):
    B = y.shape[0]
    bb = 200
    grid = (B // bb,)
    mu3 = out_mu.reshape(K, B, L)
    sg3 = out_sigma.reshape(K, B, L)
    l13 = loc1.reshape(K, B, L)
    sc3 = scale1.reshape(K, B, L)
    y2 = y.reshape(B, L)

    big = pl.BlockSpec((K, bb, L), lambda j: (0, j, 0))
    row = pl.BlockSpec((bb, L), lambda j: (j, 0))
    pis = pl.BlockSpec((bb, K), lambda j: (j, 0))
    acc = pl.BlockSpec((1, 128), lambda j: (0, 0))

    ade1, fde1, ade2, fde2, sums = pl.pallas_call(
        _body,
        grid=grid,
        in_specs=[big, big, big, big, row, pis],
        out_specs=[row, row, row, row, acc],
        out_shape=[
            jax.ShapeDtypeStruct((B, L), jnp.float32),
            jax.ShapeDtypeStruct((B, L), jnp.float32),
            jax.ShapeDtypeStruct((B, L), jnp.float32),
            jax.ShapeDtypeStruct((B, L), jnp.float32),
            jax.ShapeDtypeStruct((1, 128), jnp.float32),
        ],
    )(mu3, sg3, l13, sc3, y2, out_pi)

    reg1 = sums[0, 0] / (B * L)
    cls = -sums[0, 1] / B
    reg2 = sums[0, 2] / (B * L)
    loss0 = reg1 + cls
    loss1 = reg2

    def tra(x):
        return jnp.concatenate(
            [pre_obs, jnp.transpose(x.reshape(B, T, 2), (1, 0, 2))], axis=0)

    return (loss0, loss1, tra(ade1), tra(fde1), tra(ade2), tra(fde2))


# speedup vs baseline: 5.6425x; 1.5020x over previous
import functools

import jax
import jax.numpy as jnp
import numpy as np
from jax import lax
from jax.experimental import pallas as pl
from jax.experimental.pallas import tpu as pltpu

EPS = 1e-06
PRED_LENGTH = 12
K = 20
T = 12
L = 2 * T
B = 10000
BP = 10240
BBA = 1024
BBC = 200

def _stats_body(mu_ref, l1_ref, y_ref, pi_ref,
                i1_ref, if1_ref, i2_ref, if2_ref, sums_ref):
    j = pl.program_id(0)
    bb = y_ref.shape[1]
    y = y_ref[...]
    kio = lax.broadcasted_iota(jnp.int32, (K, bb), 0)
    bglob = j * bb + lax.broadcasted_iota(jnp.int32, (1, bb), 1)
    even = (lax.broadcasted_iota(jnp.int32, (1, L, bb), 1) % 2 == 0
            ).astype(jnp.float32)

    def head(tr):
        diff = tr - y[None]
        e = diff * diff
        s = e + pltpu.roll(e, shift=L - 1, axis=1)
        d = jnp.sqrt(s) * even
        l2 = jnp.sum(d, axis=1)
        f = jnp.sqrt(s[:, L - 2, :])
        mn = jnp.min(l2, axis=0, keepdims=True)
        am = jnp.min(jnp.where(l2 == mn, kio, K), axis=0, keepdims=True)
        mnf = jnp.min(f, axis=0, keepdims=True)
        amf = jnp.min(jnp.where(f == mnf, kio, K), axis=0, keepdims=True)
        return l2, am, amf

    l2_1, am1, amf1 = head(mu_ref[...])
    _, am2, amf2 = head(l1_ref[...])

    z = -l2_1 / PRED_LENGTH
    z = z - jnp.max(z, axis=0, keepdims=True)
    ez = jnp.exp(z)
    st = ez / jnp.sum(ez, axis=0, keepdims=True)
    pi = pi_ref[...]
    pim = pi - jnp.max(pi, axis=0, keepdims=True)
    lsm = pim - jnp.log(jnp.sum(jnp.exp(pim), axis=0, keepdims=True))
    cls_b = jnp.sum(st * lsm, axis=0, keepdims=True)
    tr_sum = jnp.sum(jnp.where(bglob < B, cls_b, 0.0))

    i1_ref[...] = am1 * B + bglob
    if1_ref[...] = amf1 * B + bglob
    i2_ref[...] = am2 * B + bglob
    if2_ref[...] = amf2 * B + bglob

    lane1 = lax.broadcasted_iota(jnp.int32, (1, 128), 1)

    @pl.when(j == 0)
    def _():
        sums_ref[...] = jnp.zeros_like(sums_ref)

    sums_ref[...] += jnp.where(lane1 == 0, tr_sum, 0.0)


def _nll_body(y_ref, m1_ref, s1_ref, m2_ref, s2_ref, sums_ref):
    j = pl.program_id(0)
    y = y_ref[...]

    def reg(m, s):
        sc = jnp.maximum(s, EPS)
        return jnp.sum(jnp.log(2.0 * sc) + jnp.abs(y - m) / sc)

    r1 = reg(m1_ref[...], s1_ref[...])
    r2 = reg(m2_ref[...], s2_ref[...])
    lane1 = lax.broadcasted_iota(jnp.int32, (1, 128), 1)

    @pl.when(j == 0)
    def _():
        sums_ref[...] = jnp.zeros_like(sums_ref)

    sums_ref[...] += (jnp.where(lane1 == 0, r1, 0.0)
                      + jnp.where(lane1 == 1, r2, 0.0))


def _gather_rows(tables, idxs):
    outs = []
    for tab, idx in zip(tables, idxs):
        outs.append(jnp.take(tab, idx, axis=0))
    return outs


@jax.jit
def kernel(out_mu, out_sigma, out_pi, loc1, scale1, y, pre_obs):
    mu2d = out_mu.reshape(K * B, L)
    sg2d = out_sigma.reshape(K * B, L)
    l12d = loc1.reshape(K * B, L)
    sc2d = scale1.reshape(K * B, L)
    y2 = y.reshape(B, L)

    pad = [(0, 0), (0, 0), (0, BP - B)]
    muT = jnp.pad(jnp.transpose(out_mu.reshape(K, B, L), (0, 2, 1)), pad)
    l1T = jnp.pad(jnp.transpose(loc1.reshape(K, B, L), (0, 2, 1)), pad)
    yT = jnp.pad(y2.T, [(0, 0), (0, BP - B)])
    piT = jnp.pad(out_pi.T, [(0, 0), (0, BP - B)])

    grid_a = (BP // BBA,)
    big = pl.BlockSpec((K, L, BBA), lambda j: (0, 0, j))
    yrow = pl.BlockSpec((L, BBA), lambda j: (0, j))
    pirow = pl.BlockSpec((K, BBA), lambda j: (0, j))
    irow = pl.BlockSpec((1, BBA), lambda j: (0, j))
    acc = pl.BlockSpec((1, 128), lambda j: (0, 0))

    i1, if1, i2, if2, sums_a = pl.pallas_call(
        _stats_body,
        grid=grid_a,
        in_specs=[big, big, yrow, pirow],
        out_specs=[irow, irow, irow, irow, acc],
        out_shape=[
            jax.ShapeDtypeStruct((1, BP), jnp.int32),
            jax.ShapeDtypeStruct((1, BP), jnp.int32),
            jax.ShapeDtypeStruct((1, BP), jnp.int32),
            jax.ShapeDtypeStruct((1, BP), jnp.int32),
            jax.ShapeDtypeStruct((1, 128), jnp.float32),
        ],
    )(muT, l1T, yT, piT)

    i1v = i1.reshape(BP)[:B]
    if1v = if1.reshape(BP)[:B]
    i2v = i2.reshape(BP)[:B]
    if2v = if2.reshape(BP)[:B]
    g1m, g1mf, g1s, g2m, g2mf, g2s = _gather_rows(
        (mu2d, mu2d, sg2d, l12d, l12d, sc2d),
        (i1v, if1v, i1v, i2v, if2v, i2v))

    grid_c = (B // BBC,)
    row = pl.BlockSpec((BBC, L), lambda j: (j, 0))
    sums_c = pl.pallas_call(
        _nll_body,
        grid=grid_c,
        in_specs=[row, row, row, row, row],
        out_specs=pl.BlockSpec((1, 128), lambda j: (0, 0)),
        out_shape=jax.ShapeDtypeStruct((1, 128), jnp.float32),
    )(y2, g1m, g1s, g2m, g2s)

    loss0 = sums_c[0, 0] / (B * L) - sums_a[0, 0] / B
    loss1 = sums_c[0, 1] / (B * L)

    def tra(g):
        return jnp.concatenate(
            [pre_obs, jnp.transpose(g.reshape(B, T, 2), (1, 0, 2))], axis=0)

    return (loss0, loss1, tra(g1m), tra(g1mf), tra(g2m), tra(g2mf))


# speedup vs baseline: 6.2428x; 1.1064x over previous
import functools

import jax
import jax.numpy as jnp
from jax import lax
from jax.experimental import pallas as pl
from jax.experimental.pallas import tpu as pltpu
from jax.experimental.pallas import tpu_sc as plsc

EPS = 1e-06
PRED_LENGTH = 12
K = 20
T = 12
L = 2 * T
B = 10000
BP = 10240
BBA = 256


def _main_body(mu_ref, sg_ref, l1_ref, sc_ref, y_ref, pi_ref,
               s1_ref, sf1_ref, s2_ref, sf2_ref, sums_ref):
    j = pl.program_id(0)
    bb = y_ref.shape[0]
    yt = lax.transpose(y_ref[...], (1, 0))
    kio = lax.broadcasted_iota(jnp.int32, (K, bb), 0)
    bglob = j * bb + lax.broadcasted_iota(jnp.int32, (1, bb), 1)
    valid = bglob < B
    validc = lax.transpose(valid, (1, 0))
    even = (lax.broadcasted_iota(jnp.int32, (L, bb), 0) % 2 == 0
            ).astype(jnp.float32)

    def head(ref, sgref):
        tks, l2s, fs = [], [], []
        for k in range(K):
            tk = lax.transpose(ref[k], (1, 0))
            diff = tk - yt
            e = diff * diff
            s = e + pltpu.roll(e, shift=L - 1, axis=0)
            d = jnp.sqrt(s) * even
            tks.append(tk)
            l2s.append(jnp.sum(d, axis=0, keepdims=True))
            fs.append(jnp.sqrt(s[L - 2:L - 1, :]))
        l2 = jnp.concatenate(l2s, axis=0)
        f = jnp.concatenate(fs, axis=0)
        mn = jnp.min(l2, axis=0, keepdims=True)
        am = jnp.min(jnp.where(l2 == mn, kio, K), axis=0, keepdims=True)
        mnf = jnp.min(f, axis=0, keepdims=True)
        amf = jnp.min(jnp.where(f == mnf, kio, K), axis=0, keepdims=True)

        mu_ade = jnp.zeros_like(tks[0])
        mu_fde = jnp.zeros_like(tks[0])
        for k in range(K):
            mu_ade = mu_ade + jnp.where(am == k, tks[k], 0.0)
            mu_fde = mu_fde + jnp.where(amf == k, tks[k], 0.0)

        am_col = lax.transpose(am, (1, 0))
        sg_ade = jnp.zeros((bb, L), jnp.float32)
        for k in range(K):
            sg_ade = sg_ade + jnp.where(am_col == k, sgref[k], 0.0)
        sg_t = lax.transpose(sg_ade, (1, 0))

        sc = jnp.maximum(sg_t, EPS)
        nll = jnp.log(2.0 * sc) + jnp.abs(yt - mu_ade) / sc
        reg = jnp.sum(jnp.where(valid, nll, 0.0))
        return l2, mu_ade, mu_fde, reg

    l2_1, a1, f1, reg1 = head(mu_ref, sg_ref)
    _, a2, f2, reg2 = head(l1_ref, sc_ref)

    z = -l2_1 / PRED_LENGTH
    z = z - jnp.max(z, axis=0, keepdims=True)
    ez = jnp.exp(z)
    st = ez / jnp.sum(ez, axis=0, keepdims=True)
    pi = lax.transpose(pi_ref[...], (1, 0))
    pim = pi - jnp.max(pi, axis=0, keepdims=True)
    lsm = pim - jnp.log(jnp.sum(jnp.exp(pim), axis=0, keepdims=True))
    cls_b = jnp.sum(st * lsm, axis=0, keepdims=True)
    tr_sum = jnp.sum(jnp.where(valid, cls_b, 0.0))

    s1_ref[...] = a1
    sf1_ref[...] = f1
    s2_ref[...] = a2
    sf2_ref[...] = f2

    lane1 = lax.broadcasted_iota(jnp.int32, (1, 128), 1)

    @pl.when(j == 0)
    def _():
        sums_ref[...] = jnp.zeros_like(sums_ref)

    sums_ref[...] += (jnp.where(lane1 == 0, tr_sum, 0.0)
                      + jnp.where(lane1 == 1, reg1, 0.0)
                      + jnp.where(lane1 == 2, reg2, 0.0))


@jax.jit
def kernel(out_mu, out_sigma, out_pi, loc1, scale1, y, pre_obs):
    mu3 = out_mu.reshape(K, B, L)
    sg3 = out_sigma.reshape(K, B, L)
    l13 = loc1.reshape(K, B, L)
    sc3 = scale1.reshape(K, B, L)
    y2 = y.reshape(B, L)

    grid = (BP // BBA,)
    big = pl.BlockSpec((K, BBA, L), lambda j: (0, j, 0))
    yrow = pl.BlockSpec((BBA, L), lambda j: (j, 0))
    pirow = pl.BlockSpec((BBA, K), lambda j: (j, 0))
    srow = pl.BlockSpec((L, BBA), lambda j: (0, j))
    acc = pl.BlockSpec((1, 128), lambda j: (0, 0))

    s1, sf1, s2, sf2, sums = pl.pallas_call(
        _main_body,
        grid=grid,
        in_specs=[big, big, big, big, yrow, pirow],
        out_specs=[srow, srow, srow, srow, acc],
        out_shape=[
            jax.ShapeDtypeStruct((L, BP), jnp.float32),
            jax.ShapeDtypeStruct((L, BP), jnp.float32),
            jax.ShapeDtypeStruct((L, BP), jnp.float32),
            jax.ShapeDtypeStruct((L, BP), jnp.float32),
            jax.ShapeDtypeStruct((1, 128), jnp.float32),
        ],
    )(mu3, sg3, l13, sc3, y2, out_pi)

    loss0 = sums[0, 1] / (B * L) - sums[0, 0] / B
    loss1 = sums[0, 2] / (B * L)

    def tra(s):
        samp = jnp.transpose(s[:, :B].reshape(T, 2, B), (0, 2, 1))
        return jnp.concatenate([pre_obs, samp], axis=0)

    return (loss0, loss1, tra(s1), tra(sf1), tra(s2), tra(sf2))
